# baseline (speedup 1.0000x reference)
import functools

import jax
import jax.numpy as jnp
from jax import lax
from jax.experimental import pallas as pl
from jax.experimental.pallas import tpu as pltpu
from jax.experimental.pallas import tpu_sc as plsc

_N_ROWS = 1000000
_BATCH = 16384
_DIM = 64
_PAD = 128
_NUM_CORES = 2
_NUM_SUBCORES = 16
_NUM_WORKERS = _NUM_CORES * _NUM_SUBCORES
_N_BKT = 7813
_BKT_PER_W = 245
_BKT_PAD = 248
_DEPTH = 8
_SROWS = 48
_OUT_ROWS = _BATCH + 64


def _make_gather():
  mesh = plsc.VectorSubcoreMesh(core_axis_name="c", subcore_axis_name="s")

  @functools.partial(
      pl.kernel,
      mesh=mesh,
      out_type=jax.ShapeDtypeStruct((_OUT_ROWS, _PAD), jnp.float32),
      scratch_types=[
          pltpu.VMEM((_BATCH,), jnp.int32),
          pltpu.VMEM((_BATCH + 16,), jnp.int32),
          pltpu.VMEM((_BATCH + 16,), jnp.int32),
          pltpu.VMEM((_DEPTH, _DIM, _PAD), jnp.float32),
          pltpu.VMEM((2, _SROWS, _PAD), jnp.float32),
          pltpu.VMEM((2, 64), jnp.int32),
      ] + [pltpu.SemaphoreType.DMA] * (_DEPTH + 2),
      compiler_params=pltpu.CompilerParams(
          use_tc_tiling_on_sc=True, needs_layout_passes=False),
  )
  def gather_kernel(tablet_hbm, idx_hbm, outp_hbm,
                    idx_v, rpk, jpk, grp, sbuf, jscat, *sems_all):
    sg = sems_all[:_DEPTH]
    sf = sems_all[_DEPTH:]
    wid = lax.axis_index("s") * _NUM_CORES + lax.axis_index("c")
    iota = lax.iota(jnp.int32, 16)

    pltpu.sync_copy(idx_hbm, idx_v)

    blo = _BKT_PER_W * wid
    row_lo = blo * _PAD
    row_hi = row_lo + _BKT_PER_W * _PAD

    def bin_body(g, cnt):
      rv = idx_v[pl.ds(g * 16, 16)]
      jv = iota + g * 16
      m = jnp.logical_and(rv >= row_lo, rv < row_hi)
      plsc.store_compressed(rpk.at[pl.ds(cnt, 16)], rv, mask=m)
      plsc.store_compressed(jpk.at[pl.ds(cnt, 16)], jv, mask=m)
      return cnt + plsc.all_reduce_population_count(m)[0]

    cnt = lax.fori_loop(0, _BATCH // 16, bin_body, jnp.int32(0))
    nvec = (cnt + 15) // 16

    for p in range(2):
      for q in range(4):
        jscat[p, pl.ds(q * 16, 16)] = jnp.full((16,), _BATCH, jnp.int32)

    def live(kb):
      return jnp.logical_and(kb < _BKT_PER_W, blo + kb < _N_BKT)

    def fetch(kb, b):
      off = pl.multiple_of((blo + kb) * _PAD, _PAD)
      pltpu.async_copy(tablet_hbm.at[:, pl.ds(off, _PAD)], grp.at[b], sg[b])

    def drain(b):
      pltpu.make_async_copy(
          tablet_hbm.at[:, pl.ds(0, _PAD)], grp.at[b], sg[b]).wait()

    def flush(p, fl):
      pltpu.async_copy(
          sbuf.at[p], outp_hbm.at[jscat.at[p, pl.ds(0, _SROWS)]], sf[p])

      @pl.when(fl >= 1)
      def _():
        pltpu.make_async_copy(
            sbuf.at[1 - p], outp_hbm.at[jscat.at[1 - p, pl.ds(0, _SROWS)]],
            sf[1 - p]).wait()

      for q in range(4):
        jscat[1 - p, pl.ds(q * 16, 16)] = jnp.full((16,), _BATCH, jnp.int32)

    for b in range(_DEPTH):
      @pl.when(live(jnp.int32(b)))
      def _(b=b):
        fetch(jnp.int32(b), b)

    def round_body(t, carry):
      for half in range(1):
        base_kb = t * _DEPTH
        for i in range(_DEPTH):
          @pl.when(live(base_kb + i))
          def _(i=i):
            drain(i)

        group_lo = (blo + base_kb) * _PAD

        def scan_body(g, carry2, group_lo=group_lo):
          srow, par, fl = carry2
          off16 = g * 16
          rv = rpk[pl.ds(off16, 16)]
          m = jnp.logical_and(
              (iota + off16) < cnt,
              jnp.logical_and(rv >= group_lo, rv < group_lo + _DEPTH * _PAD))

          def w_cond(c):
            return plsc.all_reduce_population_count(c[0])[0] > 0

          def w_body(c):
            m, srow, par, fl = c
            l = plsc.all_reduce_ffs(m)[0]
            r = rpk[pl.ds(off16 + l, 16)][0]
            j = jpk[pl.ds(off16 + l, 16)][0]
            slot = jnp.bitwise_and(
                lax.shift_right_logical(r, 7) - blo, _DEPTH - 1)
            lv = jnp.full((16,), jnp.bitwise_and(r, _PAD - 1), jnp.int32)
            for q in range(_DIM // 16):
              cv = iota + q * 16
              vals = plsc.load_gather(grp.at[slot], [cv, lv])
              sbuf[par, srow, pl.ds(q * 16, 16)] = vals
            plsc.store_scatter(
                jscat.at[par], [jnp.full((16,), srow, jnp.int32)],
                jnp.full((16,), j, jnp.int32), mask=(iota == 0))
            srow = srow + 1
            full = srow >= _SROWS
            for p in range(2):
              @pl.when(jnp.logical_and(full, par == p))
              def _(p=p):
                flush(p, fl)
            flushed = jnp.where(full, 1, 0)
            new_par = jnp.where(full, 1 - par, par)
            new_srow = jnp.where(full, 0, srow)
            m = jnp.logical_and(m, iota != l)
            return (m, new_srow, new_par, fl + flushed)

          m, srow, par, fl = lax.while_loop(
              w_cond, w_body, (m, srow, par, fl))
          return (srow, par, fl)

        carry = lax.fori_loop(0, nvec, scan_body, carry)

        for i in range(_DEPTH):
          @pl.when(live(base_kb + _DEPTH + i))
          def _(i=i, base_kb=base_kb):
            fetch(base_kb + _DEPTH + i, i)

      return carry

    carry = lax.fori_loop(
        0, _BKT_PAD // _DEPTH, round_body,
        (jnp.int32(0), jnp.int32(0), jnp.int32(0)))
    srow, par, fl = carry

    for p in range(2):
      @pl.when(par == p)
      def _(p=p):
        pltpu.async_copy(
            sbuf.at[p], outp_hbm.at[jscat.at[p, pl.ds(0, _SROWS)]], sf[p])
        pltpu.make_async_copy(
            sbuf.at[p], outp_hbm.at[jscat.at[p, pl.ds(0, _SROWS)]],
            sf[p]).wait()

        @pl.when(fl >= 1)
        def _():
          pltpu.make_async_copy(
              sbuf.at[1 - p], outp_hbm.at[jscat.at[1 - p, pl.ds(0, _SROWS)]],
              sf[1 - p]).wait()

  return gather_kernel


_gather = _make_gather()


@jax.jit
def kernel(y_true, proxies_weight):
  padded = _gather(proxies_weight.T, y_true.astype(jnp.int32))
  return padded[:_BATCH, :_DIM]


# speedup vs baseline: 1.2741x; 1.2741x over previous
import functools

import jax
import jax.numpy as jnp
from jax import lax
from jax.experimental import pallas as pl
from jax.experimental.pallas import tpu as pltpu
from jax.experimental.pallas import tpu_sc as plsc

_N_ROWS = 1000000
_BATCH = 16384
_DIM = 64
_PAD = 128
_NUM_CORES = 2
_NUM_SUBCORES = 16
_NUM_WORKERS = _NUM_CORES * _NUM_SUBCORES
_N_BKT = 7813
_BKT_PER_W = 245
_BKT_PAD = 248
_DEPTH = 8
_SROWS = 48
_OUT_ROWS = _BATCH + 64


def _make_gather():
  mesh = plsc.VectorSubcoreMesh(core_axis_name="c", subcore_axis_name="s")

  @functools.partial(
      pl.kernel,
      mesh=mesh,
      out_type=jax.ShapeDtypeStruct((_OUT_ROWS, _PAD), jnp.float32),
      scratch_types=[
          pltpu.VMEM((_BATCH,), jnp.int32),
          pltpu.VMEM((_BATCH + 16,), jnp.int32),
          pltpu.VMEM((_BATCH + 16,), jnp.int32),
          pltpu.VMEM((_DEPTH, _DIM, _PAD), jnp.float32),
          pltpu.VMEM((2, _SROWS, _PAD), jnp.float32),
          pltpu.VMEM((2, 64), jnp.int32),
      ] + [pltpu.SemaphoreType.DMA] * (_DEPTH + 2),
      compiler_params=pltpu.CompilerParams(
          use_tc_tiling_on_sc=True, needs_layout_passes=False),
  )
  def gather_kernel(tablet_hbm, idx_hbm, outp_hbm,
                    idx_v, rpk, jpk, grp, sbuf, jscat, *sems_all):
    sg = sems_all[:_DEPTH]
    sf = sems_all[_DEPTH:]
    wid = lax.axis_index("s") * _NUM_CORES + lax.axis_index("c")
    iota = lax.iota(jnp.int32, 16)

    pltpu.sync_copy(idx_hbm, idx_v)

    blo = _BKT_PER_W * wid
    row_lo = blo * _PAD
    row_hi = row_lo + _BKT_PER_W * _PAD

    def bin_body(g, cnt):
      rv = idx_v[pl.ds(g * 16, 16)]
      jv = iota + g * 16
      m = jnp.logical_and(rv >= row_lo, rv < row_hi)
      plsc.store_compressed(rpk.at[pl.ds(cnt, 16)], rv, mask=m)
      plsc.store_compressed(jpk.at[pl.ds(cnt, 16)], jv, mask=m)
      return cnt + plsc.all_reduce_population_count(m)[0]

    cnt = lax.fori_loop(0, _BATCH // 16, bin_body, jnp.int32(0))
    nvec = (cnt + 15) // 16

    for p in range(2):
      for q in range(4):
        jscat[p, pl.ds(q * 16, 16)] = jnp.full((16,), _BATCH, jnp.int32)

    def live(kb):
      return jnp.logical_and(kb < _BKT_PER_W, blo + kb < _N_BKT)

    def fetch(kb, b):
      off = pl.multiple_of((blo + kb) * _PAD, _PAD)
      pltpu.async_copy(tablet_hbm.at[:, pl.ds(off, _PAD)], grp.at[b], sg[b])

    def drain(b):
      pltpu.make_async_copy(
          tablet_hbm.at[:, pl.ds(0, _PAD)], grp.at[b], sg[b]).wait()

    def flush(p, fl):
      pltpu.async_copy(
          sbuf.at[p], outp_hbm.at[jscat.at[p, pl.ds(0, _SROWS)]], sf[p])

      @pl.when(fl >= 1)
      def _():
        pltpu.make_async_copy(
            sbuf.at[1 - p], outp_hbm.at[jscat.at[1 - p, pl.ds(0, _SROWS)]],
            sf[1 - p]).wait()

      for q in range(4):
        jscat[1 - p, pl.ds(q * 16, 16)] = jnp.full((16,), _BATCH, jnp.int32)

    for b in range(_DEPTH):
      @pl.when(live(jnp.int32(b)))
      def _(b=b):
        fetch(jnp.int32(b), b)

    def round_body(t, carry):
      for half in range(2):
        base_kb = t * _DEPTH + half * 4
        for i in range(4):
          @pl.when(live(base_kb + i))
          def _(half=half, i=i):
            drain(half * 4 + i)

        group_lo = (blo + base_kb) * _PAD

        def scan_body(g, carry2, group_lo=group_lo):
          srow, par, fl = carry2
          off16 = g * 16
          rv = rpk[pl.ds(off16, 16)]
          m = jnp.logical_and(
              (iota + off16) < cnt,
              jnp.logical_and(rv >= group_lo, rv < group_lo + 4 * _PAD))

          def w_cond(c):
            return plsc.all_reduce_population_count(c[0])[0] > 0

          def w_body(c):
            m, srow, par, fl = c
            l = plsc.all_reduce_ffs(m)[0]
            r = rpk[pl.ds(off16 + l, 16)][0]
            j = jpk[pl.ds(off16 + l, 16)][0]
            slot = jnp.bitwise_and(
                lax.shift_right_logical(r, 7) - blo, _DEPTH - 1)
            lv = jnp.full((16,), jnp.bitwise_and(r, _PAD - 1), jnp.int32)
            for q in range(_DIM // 16):
              cv = iota + q * 16
              vals = plsc.load_gather(grp.at[slot], [cv, lv])
              sbuf[par, srow, pl.ds(q * 16, 16)] = vals
            plsc.store_scatter(
                jscat.at[par], [jnp.full((16,), srow, jnp.int32)],
                jnp.full((16,), j, jnp.int32), mask=(iota == 0))
            srow = srow + 1
            full = srow >= _SROWS
            for p in range(2):
              @pl.when(jnp.logical_and(full, par == p))
              def _(p=p):
                flush(p, fl)
            flushed = jnp.where(full, 1, 0)
            new_par = jnp.where(full, 1 - par, par)
            new_srow = jnp.where(full, 0, srow)
            m = jnp.logical_and(m, iota != l)
            return (m, new_srow, new_par, fl + flushed)

          m, srow, par, fl = lax.while_loop(
              w_cond, w_body, (m, srow, par, fl))
          return (srow, par, fl)

        carry = lax.fori_loop(0, nvec, scan_body, carry)

        for i in range(4):
          @pl.when(live(base_kb + _DEPTH + i))
          def _(half=half, i=i, base_kb=base_kb):
            fetch(base_kb + _DEPTH + i, half * 4 + i)

      return carry

    carry = lax.fori_loop(
        0, _BKT_PAD // _DEPTH, round_body,
        (jnp.int32(0), jnp.int32(0), jnp.int32(0)))
    srow, par, fl = carry

    for p in range(2):
      @pl.when(par == p)
      def _(p=p):
        pltpu.async_copy(
            sbuf.at[p], outp_hbm.at[jscat.at[p, pl.ds(0, _SROWS)]], sf[p])
        pltpu.make_async_copy(
            sbuf.at[p], outp_hbm.at[jscat.at[p, pl.ds(0, _SROWS)]],
            sf[p]).wait()

        @pl.when(fl >= 1)
        def _():
          pltpu.make_async_copy(
              sbuf.at[1 - p], outp_hbm.at[jscat.at[1 - p, pl.ds(0, _SROWS)]],
              sf[1 - p]).wait()

  return gather_kernel


_gather = _make_gather()


@jax.jit
def kernel(y_true, proxies_weight):
  padded = _gather(proxies_weight.T, y_true.astype(jnp.int32))
  return padded[:_BATCH, :_DIM]
